# baseline (speedup 1.0000x reference)
import functools

import jax
import jax.numpy as jnp
from jax import lax
from jax.experimental import pallas as pl
from jax.experimental.pallas import tpu as pltpu
from jax.experimental.pallas import tpu_sc as plsc

_WIN = 32


def _make_sc_window_gather(B, N, D, chunk_base, chunk_batches):
    info = plsc.get_sparse_core_info()
    NC, NS, L = info.num_cores, info.num_subcores, info.num_lanes
    NW = NC * NS
    bpw = chunk_batches // NW
    RG = 8
    n_pairs = bpw // (2 * RG)

    mesh = plsc.VectorSubcoreMesh(core_axis_name="c", subcore_axis_name="s")

    @functools.partial(
        pl.kernel,
        mesh=mesh,
        compiler_params=pltpu.CompilerParams(
            use_tc_tiling_on_sc=True, needs_layout_passes=False),
        out_type=jax.ShapeDtypeStruct((chunk_batches, 3 * D), jnp.float32),
        scratch_types=[
            pltpu.VMEM((bpw,), jnp.int32),
            pltpu.VMEM((RG, D, 128), jnp.float32),
            pltpu.VMEM((RG - 1, D, 128), jnp.float32),
            pltpu.VMEM((RG, 3 * D), jnp.float32),
            pltpu.SemaphoreType.DMA,
        ],
    )
    def window_gather(table_hbm, pos_hbm, out_hbm, pos_v, win_v, edge_v,
                      rows_v, sem):
        wid = lax.axis_index("c") * NS + lax.axis_index("s")
        base_batch = wid * bpw
        pltpu.sync_copy(pos_hbm.at[pl.ds(base_batch, bpw)], pos_v)
        lanes = lax.iota(jnp.int32, L)

        def do_round(rb, pv, lbase):
            copies = []
            ncross = jnp.int32(0)
            crosses = []
            uppers = []
            for q in range(RG):
                lq = lbase + q
                p = jnp.max(jnp.where(lanes == lq, pv, 0))
                c0 = jnp.clip(p - 1, 0, N - 1)
                nt0 = jnp.right_shift(c0, 7)
                off0 = pl.multiple_of(jnp.left_shift(nt0, 7), 128)
                row0 = pl.multiple_of(
                    (chunk_base + base_batch + rb + q) * D, 8)
                copies.append(pltpu.async_copy(
                    table_hbm.at[pl.ds(row0, D), pl.ds(off0, 128)],
                    win_v.at[q], sem))
                c2 = jnp.clip(p + 1, 0, N - 1)
                cross = jnp.right_shift(c2, 7) > nt0
                crosses.append(cross)
                off1 = pl.multiple_of(
                    jnp.minimum(off0 + 128, N - 128), 128)
                uppers.append((row0, off1))
                if q < RG - 1:
                    ncross = ncross + cross.astype(jnp.int32)

                    @pl.when(cross)
                    def _():
                        pltpu.async_copy(
                            table_hbm.at[pl.ds(row0, D), pl.ds(off1, 128)],
                            edge_v.at[q], sem)
            for cp in copies:
                cp.wait()

            def drain_body(i, carry):
                pltpu.make_async_copy(
                    table_hbm.at[pl.ds(0, D), pl.ds(0, 128)],
                    edge_v.at[0], sem).wait()
                return carry

            lax.fori_loop(0, ncross, drain_body, 0)
            for q in range(RG):
                lq = lbase + q
                p = jnp.max(jnp.where(lanes == lq, pv, 0))
                c0 = jnp.clip(p - 1, 0, N - 1)
                off0 = jnp.left_shift(jnp.right_shift(c0, 7), 7)
                qv = jnp.full((L,), q, jnp.int32)
                esl = q
                if q == RG - 1:
                    esl = 0

                    @pl.when(crosses[q])
                    def _():
                        row0l, off1l = uppers[q]
                        pltpu.sync_copy(
                            table_hbm.at[pl.ds(row0l, D), pl.ds(off1l, 128)],
                            edge_v.at[0])
                ev_q = jnp.full((L,), esl, jnp.int32)
                for k in range(3):
                    ck = jnp.clip(p + (k - 1), 0, N - 1)
                    w = ck - off0
                    hv = jnp.full((L,), (w >= 128).astype(jnp.int32),
                                  jnp.int32) > 0
                    wv = jnp.full((L,), jnp.minimum(w, 127), jnp.int32)
                    ev = jnp.full((L,), jnp.clip(w - 128, 0, 127), jnp.int32)
                    for g in range(D // L):
                        dv = g * L + lanes
                        lo_vals = plsc.load_gather(win_v, [qv, dv, wv])
                        hi_vals = plsc.load_gather(edge_v, [ev_q, dv, ev])
                        rows_v[q, pl.ds(k * D + g * L, L)] = jnp.where(
                            hv, hi_vals, lo_vals)
            dst0 = pl.multiple_of(base_batch + rb, 8)
            pltpu.sync_copy(rows_v, out_hbm.at[pl.ds(dst0, RG)])

        def pair_body(pair, carry):
            rb = pair * (2 * RG)
            pv = pos_v[pl.ds(pl.multiple_of(rb, 2 * RG), L)]
            do_round(rb, pv, 0)
            do_round(rb + RG, pv, RG)
            return carry

        lax.fori_loop(0, n_pairs, pair_body, 0)

    return window_gather


def _project_block(e_ref, st_ref, wt_ref, bt_ref, out_ref, *, D, OUT):
    bias3t = lax.dot_general(
        wt_ref[D:, :], st_ref[...], (((0,), (1,)), ((), ())),
        precision=lax.Precision.HIGHEST,
        preferred_element_type=jnp.float32) + bt_ref[...]
    w1 = wt_ref[0:D, :]
    for k in range(3):
        ek = e_ref[:, k * D:(k + 1) * D]
        ok = lax.dot_general(
            w1, ek, (((0,), (1,)), ((), ())),
            precision=lax.Precision.HIGHEST,
            preferred_element_type=jnp.float32) + bias3t[:, k:k + 1]
        out_ref[pl.ds(k * OUT, OUT), :] = ok


def kernel(codon_embeddings, site_positions, site_table, W, b):
    B, N, D = codon_embeddings.shape
    OUT = W.shape[0]
    SITE = W.shape[1] - D

    table = jnp.swapaxes(codon_embeddings, 1, 2).reshape(B * D, N)
    pos = site_positions.astype(jnp.int32)

    st_sel = jnp.stack([site_table[2], site_table[0], site_table[1]])
    wt = W.T
    b2t = b.reshape(OUT, 1)

    n_chunks = 2
    half = B // n_chunks
    blk = 512
    outs = []
    for c in range(n_chunks):
        pos_c = lax.slice(pos, (c * half,), ((c + 1) * half,))
        erows = _make_sc_window_gather(B, N, D, c * half, half)(table, pos_c)
        outs.append(pl.pallas_call(
            functools.partial(_project_block, D=D, OUT=OUT),
            grid=(half // blk,),
            in_specs=[
                pl.BlockSpec((blk, 3 * D), lambda i: (i, 0)),
                pl.BlockSpec((3, SITE), lambda i: (0, 0)),
                pl.BlockSpec((D + SITE, OUT), lambda i: (0, 0)),
                pl.BlockSpec((OUT, 1), lambda i: (0, 0)),
            ],
            out_specs=pl.BlockSpec((3 * OUT, blk), lambda i: (0, i)),
            out_shape=jax.ShapeDtypeStruct((3 * OUT, half), jnp.float32),
        )(erows, st_sel, wt, b2t))

    out = jnp.concatenate(outs, axis=1)
    return out.reshape(3, OUT, B).transpose(2, 0, 1)


# speedup vs baseline: 1.0843x; 1.0843x over previous
import functools

import jax
import jax.numpy as jnp
from jax import lax
from jax.experimental import pallas as pl
from jax.experimental.pallas import tpu as pltpu
from jax.experimental.pallas import tpu_sc as plsc


def _make_sc_window_gather(B, N, D):
    info = plsc.get_sparse_core_info()
    NC, NS, L = info.num_cores, info.num_subcores, info.num_lanes
    NW = NC * NS
    bpw = B // NW
    RG = 8
    n_pairs = bpw // (2 * RG)

    mesh = plsc.VectorSubcoreMesh(core_axis_name="c", subcore_axis_name="s")

    @functools.partial(
        pl.kernel,
        mesh=mesh,
        compiler_params=pltpu.CompilerParams(
            use_tc_tiling_on_sc=True, needs_layout_passes=False),
        out_type=jax.ShapeDtypeStruct((B, 3 * D), jnp.float32),
        scratch_types=[
            pltpu.VMEM((bpw,), jnp.int32),
            pltpu.VMEM((RG, D, 128), jnp.float32),
            pltpu.VMEM((RG - 1, D, 128), jnp.float32),
            pltpu.VMEM((RG, 3 * D), jnp.float32),
            pltpu.SemaphoreType.DMA,
        ],
    )
    def window_gather(table_hbm, pos_hbm, out_hbm, pos_v, win_v, edge_v,
                      rows_v, sem):
        wid = lax.axis_index("c") * NS + lax.axis_index("s")
        base_batch = wid * bpw
        pltpu.sync_copy(pos_hbm.at[pl.ds(base_batch, bpw)], pos_v)
        lanes = lax.iota(jnp.int32, L)

        def do_round(rb, pv, lbase):
            copies = []
            ncross = jnp.int32(0)
            crosses = []
            uppers = []
            for q in range(RG):
                lq = lbase + q
                p = jnp.max(jnp.where(lanes == lq, pv, 0))
                c0 = jnp.clip(p - 1, 0, N - 1)
                nt0 = jnp.right_shift(c0, 7)
                off0 = pl.multiple_of(jnp.left_shift(nt0, 7), 128)
                row0 = pl.multiple_of((base_batch + rb + q) * D, 8)
                copies.append(pltpu.async_copy(
                    table_hbm.at[pl.ds(row0, D), pl.ds(off0, 128)],
                    win_v.at[q], sem))
                c2 = jnp.clip(p + 1, 0, N - 1)
                cross = jnp.right_shift(c2, 7) > nt0
                crosses.append(cross)
                off1 = pl.multiple_of(
                    jnp.minimum(off0 + 128, N - 128), 128)
                uppers.append((row0, off1))
                if q < RG - 1:
                    ncross = ncross + cross.astype(jnp.int32)

                    @pl.when(cross)
                    def _():
                        pltpu.async_copy(
                            table_hbm.at[pl.ds(row0, D), pl.ds(off1, 128)],
                            edge_v.at[q], sem)
            for cp in copies:
                cp.wait()

            def drain_body(i, carry):
                pltpu.make_async_copy(
                    table_hbm.at[pl.ds(0, D), pl.ds(0, 128)],
                    edge_v.at[0], sem).wait()
                return carry

            lax.fori_loop(0, ncross, drain_body, 0)
            for q in range(RG):
                lq = lbase + q
                p = jnp.max(jnp.where(lanes == lq, pv, 0))
                c0 = jnp.clip(p - 1, 0, N - 1)
                off0 = jnp.left_shift(jnp.right_shift(c0, 7), 7)
                qv = jnp.full((L,), q, jnp.int32)
                esl = q
                if q == RG - 1:
                    esl = 0

                    @pl.when(crosses[q])
                    def _():
                        row0l, off1l = uppers[q]
                        pltpu.sync_copy(
                            table_hbm.at[pl.ds(row0l, D), pl.ds(off1l, 128)],
                            edge_v.at[0])
                ev_q = jnp.full((L,), esl, jnp.int32)
                for k in range(3):
                    ck = jnp.clip(p + (k - 1), 0, N - 1)
                    w = ck - off0
                    hv = jnp.full((L,), (w >= 128).astype(jnp.int32),
                                  jnp.int32) > 0
                    wv = jnp.full((L,), jnp.minimum(w, 127), jnp.int32)
                    ev = jnp.full((L,), jnp.clip(w - 128, 0, 127), jnp.int32)
                    for g in range(D // L):
                        dv = g * L + lanes
                        lo_vals = plsc.load_gather(win_v, [qv, dv, wv])
                        hi_vals = plsc.load_gather(edge_v, [ev_q, dv, ev])
                        rows_v[q, pl.ds(k * D + g * L, L)] = jnp.where(
                            hv, hi_vals, lo_vals)
            dst0 = pl.multiple_of(base_batch + rb, 8)
            pltpu.sync_copy(rows_v, out_hbm.at[pl.ds(dst0, RG)])

        def pair_body(pair, carry):
            rb = pair * (2 * RG)
            pv = pos_v[pl.ds(pl.multiple_of(rb, 2 * RG), L)]
            do_round(rb, pv, 0)
            do_round(rb + RG, pv, RG)
            return carry

        lax.fori_loop(0, n_pairs, pair_body, 0)

    return window_gather


def _project_block(e_ref, st_ref, wt_ref, bt_ref, out_ref, *, D, OUT):
    bias3t = lax.dot_general(
        wt_ref[D:, :], st_ref[...], (((0,), (1,)), ((), ())),
        precision=lax.Precision.HIGHEST,
        preferred_element_type=jnp.float32) + bt_ref[...]
    w1 = wt_ref[0:D, :]
    for k in range(3):
        ek = e_ref[:, k * D:(k + 1) * D]
        ok = lax.dot_general(
            w1, ek, (((0,), (1,)), ((), ())),
            precision=lax.Precision.HIGHEST,
            preferred_element_type=jnp.float32) + bias3t[:, k:k + 1]
        out_ref[pl.ds(k * OUT, OUT), :] = ok


def kernel(codon_embeddings, site_positions, site_table, W, b):
    B, N, D = codon_embeddings.shape
    OUT = W.shape[0]
    SITE = W.shape[1] - D

    table = jnp.swapaxes(codon_embeddings, 1, 2).reshape(B * D, N)
    pos = site_positions.astype(jnp.int32)
    erows = _make_sc_window_gather(B, N, D)(table, pos)

    st_sel = jnp.stack([site_table[2], site_table[0], site_table[1]])
    wt = W.T
    b2t = b.reshape(OUT, 1)

    blk = 512
    grid = B // blk
    out = pl.pallas_call(
        functools.partial(_project_block, D=D, OUT=OUT),
        grid=(grid,),
        in_specs=[
            pl.BlockSpec((blk, 3 * D), lambda i: (i, 0)),
            pl.BlockSpec((3, SITE), lambda i: (0, 0)),
            pl.BlockSpec((D + SITE, OUT), lambda i: (0, 0)),
            pl.BlockSpec((OUT, 1), lambda i: (0, 0)),
        ],
        out_specs=pl.BlockSpec((3 * OUT, blk), lambda i: (0, i)),
        out_shape=jax.ShapeDtypeStruct((3 * OUT, B), jnp.float32),
    )(erows, st_sel, wt, b2t)

    return out.reshape(3, OUT, B).transpose(2, 0, 1)


# speedup vs baseline: 1.1043x; 1.0184x over previous
import functools

import jax
import jax.numpy as jnp
from jax import lax
from jax.experimental import pallas as pl
from jax.experimental.pallas import tpu as pltpu
from jax.experimental.pallas import tpu_sc as plsc


def _make_sc_window_gather(B, N, D):
    info = plsc.get_sparse_core_info()
    NC, NS, L = info.num_cores, info.num_subcores, info.num_lanes
    NW = NC * NS
    bpw = B // NW
    RG = 8
    n_pairs = bpw // (2 * RG)

    mesh = plsc.VectorSubcoreMesh(core_axis_name="c", subcore_axis_name="s")

    @functools.partial(
        pl.kernel,
        mesh=mesh,
        compiler_params=pltpu.CompilerParams(
            use_tc_tiling_on_sc=True, needs_layout_passes=False),
        out_type=jax.ShapeDtypeStruct((B, 3 * D), jnp.float32),
        scratch_types=[
            pltpu.VMEM((bpw,), jnp.int32),
            pltpu.VMEM((RG, D, 128), jnp.float32),
            pltpu.VMEM((RG - 1, D, 128), jnp.float32),
            pltpu.VMEM((2, RG, 3 * D), jnp.float32),
            pltpu.SemaphoreType.DMA,
            pltpu.SemaphoreType.DMA,
            pltpu.SemaphoreType.DMA,
        ],
    )
    def window_gather(table_hbm, pos_hbm, out_hbm, pos_v, win_v, edge_v,
                      rows_v, sem, sem_a, sem_b):
        wid = lax.axis_index("c") * NS + lax.axis_index("s")
        base_batch = wid * bpw
        pltpu.sync_copy(pos_hbm.at[pl.ds(base_batch, bpw)], pos_v)
        lanes = lax.iota(jnp.int32, L)

        def do_round(rb, pv, lbase, par, fsem):
            copies = []
            ncross = jnp.int32(0)
            crosses = []
            uppers = []
            for q in range(RG):
                lq = lbase + q
                p = jnp.max(jnp.where(lanes == lq, pv, 0))
                c0 = jnp.clip(p - 1, 0, N - 1)
                nt0 = jnp.right_shift(c0, 7)
                off0 = pl.multiple_of(jnp.left_shift(nt0, 7), 128)
                row0 = pl.multiple_of((base_batch + rb + q) * D, 8)
                copies.append(pltpu.async_copy(
                    table_hbm.at[pl.ds(row0, D), pl.ds(off0, 128)],
                    win_v.at[q], sem))
                c2 = jnp.clip(p + 1, 0, N - 1)
                cross = jnp.right_shift(c2, 7) > nt0
                crosses.append(cross)
                off1 = pl.multiple_of(
                    jnp.minimum(off0 + 128, N - 128), 128)
                uppers.append((row0, off1))
                if q < RG - 1:
                    ncross = ncross + cross.astype(jnp.int32)

                    @pl.when(cross)
                    def _():
                        pltpu.async_copy(
                            table_hbm.at[pl.ds(row0, D), pl.ds(off1, 128)],
                            edge_v.at[q], sem)
            for cp in copies:
                cp.wait()

            def drain_body(i, carry):
                pltpu.make_async_copy(
                    table_hbm.at[pl.ds(0, D), pl.ds(0, 128)],
                    edge_v.at[0], sem).wait()
                return carry

            lax.fori_loop(0, ncross, drain_body, 0)
            for q in range(RG):
                lq = lbase + q
                p = jnp.max(jnp.where(lanes == lq, pv, 0))
                c0 = jnp.clip(p - 1, 0, N - 1)
                off0 = jnp.left_shift(jnp.right_shift(c0, 7), 7)
                qv = jnp.full((L,), q, jnp.int32)
                esl = q
                if q == RG - 1:
                    esl = 0

                    @pl.when(crosses[q])
                    def _():
                        row0l, off1l = uppers[q]
                        pltpu.sync_copy(
                            table_hbm.at[pl.ds(row0l, D), pl.ds(off1l, 128)],
                            edge_v.at[0])
                ev_q = jnp.full((L,), esl, jnp.int32)
                for k in range(3):
                    ck = jnp.clip(p + (k - 1), 0, N - 1)
                    w = ck - off0
                    hv = jnp.full((L,), (w >= 128).astype(jnp.int32),
                                  jnp.int32) > 0
                    wv = jnp.full((L,), jnp.minimum(w, 127), jnp.int32)
                    ev = jnp.full((L,), jnp.clip(w - 128, 0, 127), jnp.int32)
                    for g in range(D // L):
                        dv = g * L + lanes
                        lo_vals = plsc.load_gather(win_v, [qv, dv, wv])
                        hi_vals = plsc.load_gather(edge_v, [ev_q, dv, ev])
                        rows_v[par, q, pl.ds(k * D + g * L, L)] = jnp.where(
                            hv, hi_vals, lo_vals)
            dst0 = pl.multiple_of(base_batch + rb, 8)
            pltpu.async_copy(rows_v.at[par], out_hbm.at[pl.ds(dst0, RG)],
                             fsem)

        def flush_wait(par, fsem):
            pltpu.make_async_copy(
                rows_v.at[par], out_hbm.at[pl.ds(base_batch, RG)],
                fsem).wait()

        def pair_body(pair, carry):
            rb = pair * (2 * RG)
            pv = pos_v[pl.ds(pl.multiple_of(rb, 2 * RG), L)]

            @pl.when(pair > 0)
            def _():
                flush_wait(0, sem_a)

            do_round(rb, pv, 0, 0, sem_a)

            @pl.when(pair > 0)
            def _():
                flush_wait(1, sem_b)

            do_round(rb + RG, pv, RG, 1, sem_b)
            return carry

        lax.fori_loop(0, n_pairs, pair_body, 0)
        flush_wait(0, sem_a)
        flush_wait(1, sem_b)

    return window_gather


def _project_block(e_ref, st_ref, wt_ref, bt_ref, out_ref, *, D, OUT):
    bias3t = lax.dot_general(
        wt_ref[D:, :], st_ref[...], (((0,), (1,)), ((), ())),
        precision=lax.Precision.HIGHEST,
        preferred_element_type=jnp.float32) + bt_ref[...]
    w1 = wt_ref[0:D, :]
    for k in range(3):
        ek = e_ref[:, k * D:(k + 1) * D]
        ok = lax.dot_general(
            w1, ek, (((0,), (1,)), ((), ())),
            precision=lax.Precision.HIGHEST,
            preferred_element_type=jnp.float32) + bias3t[:, k:k + 1]
        out_ref[pl.ds(k * OUT, OUT), :] = ok


def kernel(codon_embeddings, site_positions, site_table, W, b):
    B, N, D = codon_embeddings.shape
    OUT = W.shape[0]
    SITE = W.shape[1] - D

    table = jnp.swapaxes(codon_embeddings, 1, 2).reshape(B * D, N)
    pos = site_positions.astype(jnp.int32)
    erows = _make_sc_window_gather(B, N, D)(table, pos)

    st_sel = jnp.stack([site_table[2], site_table[0], site_table[1]])
    wt = W.T
    b2t = b.reshape(OUT, 1)

    blk = 512
    grid = B // blk
    out = pl.pallas_call(
        functools.partial(_project_block, D=D, OUT=OUT),
        grid=(grid,),
        in_specs=[
            pl.BlockSpec((blk, 3 * D), lambda i: (i, 0)),
            pl.BlockSpec((3, SITE), lambda i: (0, 0)),
            pl.BlockSpec((D + SITE, OUT), lambda i: (0, 0)),
            pl.BlockSpec((OUT, 1), lambda i: (0, 0)),
        ],
        out_specs=pl.BlockSpec((3 * OUT, blk), lambda i: (0, i)),
        out_shape=jax.ShapeDtypeStruct((3 * OUT, B), jnp.float32),
    )(erows, st_sel, wt, b2t)

    return out.reshape(3, OUT, B).transpose(2, 0, 1)
